# baseline (speedup 1.0000x reference)
import functools

import jax
import jax.numpy as jnp
from jax import lax
from jax.experimental import pallas as pl
from jax.experimental.pallas import tpu as pltpu
from jax.experimental.pallas import tpu_sc as plsc

EMBED_DIM = 32
NUM_WORKERS = 32
IDX_MINOR = 128
STEP_ROWS = 512
SUB = STEP_ROWS // IDX_MINOR
GROUP = 16


def _rsqrt(s):
    i = plsc.bitcast(s, jnp.int32)
    i = jnp.int32(0x5F3759DF) - (i >> 1)
    y = plsc.bitcast(i, jnp.float32)
    xh = s * jnp.float32(0.5)
    for _ in range(3):
        y = y * (jnp.float32(1.5) - xh * y * y)
    return y


def _make_sc_kernel(n_rows):
    rows_per_w = n_rows // NUM_WORKERS
    n_idx_rows = rows_per_w // IDX_MINOR
    n_steps = rows_per_w // STEP_ROWS
    n_groups = STEP_ROWS // GROUP

    mesh = plsc.VectorSubcoreMesh(core_axis_name="c", subcore_axis_name="s")

    @functools.partial(
        pl.kernel,
        out_type=jax.ShapeDtypeStruct((n_rows, EMBED_DIM), jnp.float32),
        mesh=mesh,
        compiler_params=pltpu.CompilerParams(
            needs_layout_passes=False, use_tc_tiling_on_sc=False
        ),
        scratch_types=[
            pltpu.VMEM((n_idx_rows, IDX_MINOR), jnp.int32),
            pltpu.VMEM((STEP_ROWS, EMBED_DIM), jnp.float32),
            pltpu.SemaphoreType.DMA,
        ],
    )
    def sc_kernel(idx_hbm, table_hbm, out_hbm, idx_v, rows_v, gsem):
        wid = lax.axis_index("s") * 2 + lax.axis_index("c")
        pltpu.sync_copy(idx_hbm.at[wid], idx_v)
        out_base = wid * rows_per_w

        lanes = lax.iota(jnp.int32, GROUP)

        def step(s, carry):
            copies = [
                pltpu.async_copy(
                    table_hbm.at[idx_v.at[s * SUB + t]],
                    rows_v.at[pl.ds(t * IDX_MINOR, IDX_MINOR)],
                    gsem,
                )
                for t in range(SUB)
            ]
            for c in copies:
                c.wait()

            def grp(g, c):
                rvec = g * GROUP + lanes
                acc = jnp.zeros((GROUP,), jnp.float32)
                for d in range(EMBED_DIM):
                    dvec = jnp.full((GROUP,), d, jnp.int32)
                    v = plsc.load_gather(rows_v, [rvec, dvec])
                    acc = acc + v * v
                scale = _rsqrt(jnp.maximum(acc, jnp.float32(1e-24)))
                for d in range(EMBED_DIM):
                    dvec = jnp.full((GROUP,), d, jnp.int32)
                    v = plsc.load_gather(rows_v, [rvec, dvec])
                    plsc.store_scatter(rows_v, [rvec, dvec], v * scale)
                return c

            lax.fori_loop(0, n_groups, grp, 0)
            pltpu.sync_copy(
                rows_v,
                out_hbm.at[pl.ds(out_base + s * STEP_ROWS, STEP_ROWS)],
            )
            return carry

        lax.fori_loop(0, n_steps, step, 0)

    return sc_kernel


def kernel(x, table):
    batch, hist = x.shape
    n_rows = batch * hist
    rows_per_w = n_rows // NUM_WORKERS
    idx = x.astype(jnp.int32).reshape(
        NUM_WORKERS, rows_per_w // IDX_MINOR, IDX_MINOR
    )
    out = _make_sc_kernel(n_rows)(idx, table)
    return out.reshape(batch, hist, EMBED_DIM)


# speedup vs baseline: 1.3241x; 1.3241x over previous
import functools

import jax
import jax.numpy as jnp
from jax import lax
from jax.experimental import pallas as pl
from jax.experimental.pallas import tpu as pltpu
from jax.experimental.pallas import tpu_sc as plsc

EMBED_DIM = 32
HIST = 50
NUM_WORKERS = 32
IDX_PER_STREAM = 100
BATCHES_PER_STEP = 8
STEP_ROWS = BATCHES_PER_STEP * HIST
SUB = STEP_ROWS // IDX_PER_STREAM
GROUP = 16


def _rsqrt(s):
    i = plsc.bitcast(s, jnp.int32)
    i = jnp.int32(0x5F3759DF) - (i >> 1)
    y = plsc.bitcast(i, jnp.float32)
    xh = s * jnp.float32(0.5)
    for _ in range(3):
        y = y * (jnp.float32(1.5) - xh * y * y)
    return y


def _make_sc_kernel(batch):
    batches_per_w = batch // NUM_WORKERS
    rows_per_w = batches_per_w * HIST
    n_idx_rows = rows_per_w // IDX_PER_STREAM
    n_steps = batches_per_w // BATCHES_PER_STEP
    n_groups = STEP_ROWS // GROUP

    mesh = plsc.VectorSubcoreMesh(core_axis_name="c", subcore_axis_name="s")

    @functools.partial(
        pl.kernel,
        out_type=jax.ShapeDtypeStruct((batch, HIST, EMBED_DIM), jnp.float32),
        mesh=mesh,
        compiler_params=pltpu.CompilerParams(
            needs_layout_passes=False, use_tc_tiling_on_sc=False
        ),
        scratch_types=[
            pltpu.VMEM((n_idx_rows, IDX_PER_STREAM), jnp.int32),
            pltpu.VMEM((STEP_ROWS, EMBED_DIM), jnp.float32),
            pltpu.VMEM((STEP_ROWS, EMBED_DIM), jnp.float32),
            pltpu.VMEM((BATCHES_PER_STEP, HIST, EMBED_DIM), jnp.float32),
            pltpu.VMEM((BATCHES_PER_STEP, HIST, EMBED_DIM), jnp.float32),
            pltpu.SemaphoreType.DMA,
            pltpu.SemaphoreType.DMA,
            pltpu.SemaphoreType.DMA,
            pltpu.SemaphoreType.DMA,
        ],
    )
    def sc_kernel(
        idx_hbm, table_hbm, out_hbm,
        idx_v, g_a, g_b, o_a, o_b, gsem_a, gsem_b, osem_a, osem_b,
    ):
        wid = lax.axis_index("s") * 2 + lax.axis_index("c")
        pltpu.sync_copy(idx_hbm.at[wid], idx_v)
        batch_base = wid * batches_per_w
        lanes = lax.iota(jnp.int32, GROUP)

        def fire_gather(s, g_ref, gsem):
            for t in range(SUB):
                pltpu.async_copy(
                    table_hbm.at[idx_v.at[s * SUB + t]],
                    g_ref.at[pl.ds(t * IDX_PER_STREAM, IDX_PER_STREAM)],
                    gsem,
                )

        def wait_gather(s, g_ref, gsem):
            for t in range(SUB):
                pltpu.make_async_copy(
                    table_hbm.at[idx_v.at[s * SUB + t]],
                    g_ref.at[pl.ds(t * IDX_PER_STREAM, IDX_PER_STREAM)],
                    gsem,
                ).wait()

        def out_slice(s):
            return out_hbm.at[
                pl.ds(batch_base + s * BATCHES_PER_STEP, BATCHES_PER_STEP)
            ]

        def compute(g_ref, o_ref):
            def grp(g, c):
                rvec = g * GROUP + lanes
                bvec = (rvec * jnp.int32(1311)) >> 16
                hvec = rvec - jnp.int32(HIST) * bvec
                acc = jnp.zeros((GROUP,), jnp.float32)
                for d in range(EMBED_DIM):
                    dvec = jnp.full((GROUP,), d, jnp.int32)
                    v = plsc.load_gather(g_ref, [rvec, dvec])
                    acc = acc + v * v
                scale = _rsqrt(jnp.maximum(acc, jnp.float32(1e-24)))
                for d in range(EMBED_DIM):
                    dvec = jnp.full((GROUP,), d, jnp.int32)
                    v = plsc.load_gather(g_ref, [rvec, dvec])
                    plsc.store_scatter(o_ref, [bvec, hvec, dvec], v * scale)
                return c

            lax.fori_loop(0, n_groups, grp, 0)

        def do_step(s, g_ref, o_ref, gsem, osem):
            wait_gather(s, g_ref, gsem)

            @pl.when(s >= 2)
            def _():
                pltpu.make_async_copy(o_ref, out_slice(s - 2), osem).wait()

            compute(g_ref, o_ref)
            pltpu.async_copy(o_ref, out_slice(s), osem)

            @pl.when(s + 2 < n_steps)
            def _():
                fire_gather(s + 2, g_ref, gsem)

        fire_gather(0, g_a, gsem_a)
        fire_gather(1, g_b, gsem_b)

        def step(s, carry):
            @pl.when(s % 2 == 0)
            def _():
                do_step(s, g_a, o_a, gsem_a, osem_a)

            @pl.when(s % 2 == 1)
            def _():
                do_step(s, g_b, o_b, gsem_b, osem_b)

            return carry

        lax.fori_loop(0, n_steps, step, 0)
        pltpu.make_async_copy(o_a, out_slice(n_steps - 2), osem_a).wait()
        pltpu.make_async_copy(o_b, out_slice(n_steps - 1), osem_b).wait()

    return sc_kernel


def kernel(x, table):
    batch, hist = x.shape
    rows_per_w = batch * hist // NUM_WORKERS
    idx = x.astype(jnp.int32).reshape(
        NUM_WORKERS, rows_per_w // IDX_PER_STREAM, IDX_PER_STREAM
    )
    return _make_sc_kernel(batch)(idx, table)
